# baseline (speedup 1.0000x reference)
import functools

import jax
import jax.numpy as jnp
from jax import lax
from jax.experimental import pallas as pl
from jax.experimental.pallas import tpu as pltpu
from jax.experimental.pallas import tpu_sc as plsc

_CHUNKS = (128, 128, 128, 128)
_SUB = 32


def kernel(indices, logits):
    B, = indices.shape
    V, D = logits.shape
    info = plsc.get_sparse_core_info()
    NC, NS, L = info.num_cores, info.num_subcores, info.num_lanes
    NW = NC * NS
    b_per_w = B // NW
    assert sum(_CHUNKS) == b_per_w
    n_chunks = len(_CHUNKS)
    offs = [sum(_CHUNKS[:i]) for i in range(n_chunks)]
    mesh = plsc.VectorSubcoreMesh(core_axis_name="c", subcore_axis_name="s")

    @functools.partial(
        pl.kernel,
        mesh=mesh,
        out_type=jax.ShapeDtypeStruct((B, D), jnp.float32),
        scratch_types=[
            pltpu.VMEM((b_per_w,), jnp.int32),
        ] + [pltpu.VMEM((ch, D), jnp.float32) for ch in _CHUNKS]
          + [pltpu.SemaphoreType.DMA] * (2 * n_chunks),
    )
    def _run(idx_hbm, table_hbm, out_hbm, idx_v, *rest):
        bufs = rest[:n_chunks]
        gsems = rest[n_chunks:2 * n_chunks]
        wsems = rest[2 * n_chunks:3 * n_chunks]
        wid = lax.axis_index("s") * NC + lax.axis_index("c")
        base = wid * b_per_w
        pltpu.sync_copy(idx_hbm.at[pl.ds(base, b_per_w)], idx_v)

        gcopies = [
            pltpu.async_copy(
                table_hbm.at[idx_v.at[pl.ds(offs[g], _CHUNKS[g])]],
                bufs[g], gsems[g])
            for g in range(n_chunks)
        ]
        wcopies = []
        for g in range(n_chunks):
            gcopies[g].wait()
            buf = bufs[g]
            for sub in range(_CHUNKS[g] // _SUB):
                lo = sub * _SUB

                @plsc.parallel_loop(lo, lo + _SUB, unroll=4)
                def _sigmoid_rows(r):
                    for c in range(D // L):
                        x = buf[r, pl.ds(c * L, L)]
                        buf[r, pl.ds(c * L, L)] = 1.0 / (1.0 + jnp.exp(-x))

                wcopies.append(pltpu.async_copy(
                    buf.at[pl.ds(lo, _SUB)],
                    out_hbm.at[pl.ds(base + offs[g] + lo, _SUB)], wsems[g]))
        for w in wcopies:
            w.wait()

    return _run(indices, logits)


# speedup vs baseline: 1.1342x; 1.1342x over previous
import functools

import jax
import jax.numpy as jnp
from jax import lax
from jax.experimental import pallas as pl
from jax.experimental.pallas import tpu as pltpu
from jax.experimental.pallas import tpu_sc as plsc

_CHUNKS = (128, 128, 128, 128)


def kernel(indices, logits):
    B, = indices.shape
    V, D = logits.shape
    info = plsc.get_sparse_core_info()
    NC, NS, L = info.num_cores, info.num_subcores, info.num_lanes
    NW = NC * NS
    b_per_w = B // NW
    assert sum(_CHUNKS) == b_per_w
    n_chunks = len(_CHUNKS)
    offs = [sum(_CHUNKS[:i]) for i in range(n_chunks)]
    mesh = plsc.VectorSubcoreMesh(core_axis_name="c", subcore_axis_name="s")

    @functools.partial(
        pl.kernel,
        mesh=mesh,
        out_type=jax.ShapeDtypeStruct((B, D), jnp.float32),
        scratch_types=[
            pltpu.VMEM((b_per_w,), jnp.int32),
        ] + [pltpu.VMEM((ch, D), jnp.float32) for ch in _CHUNKS]
          + [pltpu.SemaphoreType.DMA] * (2 * n_chunks),
    )
    def _run(idx_hbm, table_hbm, out_hbm, idx_v, *rest):
        bufs = rest[:n_chunks]
        gsems = rest[n_chunks:2 * n_chunks]
        wsems = rest[2 * n_chunks:3 * n_chunks]
        wid = lax.axis_index("s") * NC + lax.axis_index("c")
        base = wid * b_per_w
        pltpu.sync_copy(idx_hbm.at[pl.ds(base, b_per_w)], idx_v)

        gcopies = [
            pltpu.async_copy(
                table_hbm.at[idx_v.at[pl.ds(offs[g], _CHUNKS[g])]],
                bufs[g], gsems[g])
            for g in range(n_chunks)
        ]
        wcopies = [None] * n_chunks
        for g in range(n_chunks):
            gcopies[g].wait()
            buf = bufs[g]

            @plsc.parallel_loop(0, _CHUNKS[g], unroll=4)
            def _sigmoid_rows(r):
                for c in range(D // L):
                    x = buf[r, pl.ds(c * L, L)]
                    buf[r, pl.ds(c * L, L)] = 1.0 / (1.0 + jnp.exp(-x))

            wcopies[g] = pltpu.async_copy(
                buf, out_hbm.at[pl.ds(base + offs[g], _CHUNKS[g])], wsems[g])
        for g in range(n_chunks):
            wcopies[g].wait()

    return _run(indices, logits)


# speedup vs baseline: 1.1385x; 1.0038x over previous
import functools

import jax
import jax.numpy as jnp
from jax import lax
from jax.experimental import pallas as pl
from jax.experimental.pallas import tpu as pltpu
from jax.experimental.pallas import tpu_sc as plsc

_CHUNKS = (128, 160, 160, 64)


def kernel(indices, logits):
    B, = indices.shape
    V, D = logits.shape
    info = plsc.get_sparse_core_info()
    NC, NS, L = info.num_cores, info.num_subcores, info.num_lanes
    NW = NC * NS
    b_per_w = B // NW
    assert sum(_CHUNKS) == b_per_w
    n_chunks = len(_CHUNKS)
    offs = [sum(_CHUNKS[:i]) for i in range(n_chunks)]
    mesh = plsc.VectorSubcoreMesh(core_axis_name="c", subcore_axis_name="s")

    @functools.partial(
        pl.kernel,
        mesh=mesh,
        out_type=jax.ShapeDtypeStruct((B, D), jnp.float32),
        scratch_types=[
            pltpu.VMEM((b_per_w,), jnp.int32),
        ] + [pltpu.VMEM((ch, D), jnp.float32) for ch in _CHUNKS]
          + [pltpu.SemaphoreType.DMA] * (2 * n_chunks),
    )
    def _run(idx_hbm, table_hbm, out_hbm, idx_v, *rest):
        bufs = rest[:n_chunks]
        gsems = rest[n_chunks:2 * n_chunks]
        wsems = rest[2 * n_chunks:3 * n_chunks]
        wid = lax.axis_index("s") * NC + lax.axis_index("c")
        base = wid * b_per_w
        pltpu.sync_copy(idx_hbm.at[pl.ds(base, b_per_w)], idx_v)

        gcopies = [
            pltpu.async_copy(
                table_hbm.at[idx_v.at[pl.ds(offs[g], _CHUNKS[g])]],
                bufs[g], gsems[g])
            for g in range(n_chunks)
        ]
        wcopies = [None] * n_chunks
        for g in range(n_chunks):
            gcopies[g].wait()
            buf = bufs[g]

            @plsc.parallel_loop(0, _CHUNKS[g], unroll=4)
            def _sigmoid_rows(r):
                for c in range(D // L):
                    x = buf[r, pl.ds(c * L, L)]
                    buf[r, pl.ds(c * L, L)] = 1.0 / (1.0 + jnp.exp(-x))

            wcopies[g] = pltpu.async_copy(
                buf, out_hbm.at[pl.ds(base + offs[g], _CHUNKS[g])], wsems[g])
        for g in range(n_chunks):
            wcopies[g].wait()

    return _run(indices, logits)
